# baseline (speedup 1.0000x reference)
import math

import jax
import jax.numpy as jnp
from jax.experimental import pallas as pl

_M, _K, _D = 8, 256, 32
_HW = 256
_EPS = 1e-05
_C = 0.5 * math.log(2.0 * math.pi)
_DELTA = 1.2e-4
_RCAP = 16
_NEG = -1e30
_HI = jax.lax.Precision.HIGHEST


def _tree8(w):
    parts = []
    for c in range(4):
        t = w[8 * c:8 * c + 8, :]
        t = t[0:4, :] + t[4:8, :]
        t = t[0:2, :] + t[2:4, :]
        t = t[0:1, :] + t[1:2, :]
        parts.append(t)
    return ((parts[0] + parts[1]) + parts[2]) + parts[3]


def _body(x_ref, mus_ref, musbf_ref, sc_ref, lp_ref, scr_ref, lscr_ref,
          lpr_ref, out_ref, kld_ref):
    b = pl.program_id(0)
    m = pl.program_id(1)

    xs = x_ref[0]
    mus = mus_ref[0]
    musbf = musbf_ref[0]
    sc = jnp.clip(sc_ref[0], _EPS, None)
    lp = lp_ref[0]

    a = 0.5 / (sc * sc)
    musq = jnp.sum(mus * mus, axis=1, keepdims=True)
    cst = -_D * (jnp.log(sc) + _C) + lp

    dot = jnp.dot(mus, xs, preferred_element_type=jnp.float32, precision=_HI)
    logits = a * (2.0 * dot - musq) + cst

    colmax = jnp.max(logits, axis=0, keepdims=True)
    shifted = logits - colmax
    e = jnp.exp(shifted)
    se = jnp.sum(e, axis=0, keepdims=True)
    lse_sh = jnp.log(se)
    p = e / se
    lpmax = jnp.max(lp)
    log_prior = lp - (lpmax + jnp.log(jnp.sum(jnp.exp(lp - lpmax))))
    kld_part = jnp.sum(p * (shifted - lse_sh - log_prior))

    iota = jax.lax.broadcasted_iota(jnp.int32, (_K, _HW), 0)
    lcur = logits
    idxs = []
    vals = []
    for _ in range(4):
        vmax = jnp.max(lcur, axis=0, keepdims=True)
        idx = jnp.min(jnp.where(lcur == vmax, iota, _K), axis=0, keepdims=True)
        idxs.append(idx)
        vals.append(vmax)
        lcur = jnp.where(iota == idx, _NEG, lcur)
    gap = vals[0] - vals[1]

    iota_t = jax.lax.broadcasted_iota(jnp.int32, (1, _HW), 1)
    rmask = gap < _DELTA
    tlist = []
    for _ in range(_RCAP):
        tj = jnp.min(jnp.where(rmask, iota_t, 512), axis=1, keepdims=True)
        tlist.append(tj)
        rmask = jnp.logical_and(rmask, iota_t != tj)
    tvec = jnp.concatenate(tlist, axis=1)

    iota_sub = jax.lax.broadcasted_iota(jnp.int32, (_K, _RCAP), 0)
    selT = (iota_sub == tvec).astype(jnp.float32)
    xsel = jnp.dot(xs, selT, preferred_element_type=jnp.float32,
                   precision=_HI)
    cand4 = jnp.concatenate(idxs, axis=0).astype(jnp.float32)
    codes = jnp.dot(cand4, selT, preferred_element_type=jnp.float32,
                    precision=_HI)

    scr = jnp.clip(scr_ref[0], _EPS, None)
    den_row = 2.0 * scr ** 2
    lsc_row = lscr_ref[0]
    lp_row = lpr_ref[0]

    bv = bk = None
    for cj in range(4):
        code_cj = codes[cj:cj + 1, :]
        selK = (iota_sub == code_cj).astype(jnp.float32)
        mus_cj = jax.lax.dot_general(
            mus, selK, (((0,), (0,)), ((), ())),
            preferred_element_type=jnp.float32, precision=_HI)
        den_cj = jnp.dot(den_row, selK, preferred_element_type=jnp.float32,
                         precision=_HI)
        lsc_cj = jnp.dot(lsc_row, selK, preferred_element_type=jnp.float32,
                         precision=_HI)
        lp_cj = jnp.dot(lp_row, selK, preferred_element_type=jnp.float32,
                        precision=_HI)
        diff = xsel - mus_cj
        w = -(diff * diff) / den_cj - lsc_cj - _C
        v = _tree8(w) + lp_cj
        if cj == 0:
            bv, bk = v, code_cj
        else:
            better = jnp.logical_or(
                v > bv, jnp.logical_and(v == bv, code_cj < bk))
            bv = jnp.where(better, v, bv)
            bk = jnp.where(better, code_cj, bk)
    bki = bk.astype(jnp.int32)

    idxf = idxs[0]
    for j in range(_RCAP):
        tj = tlist[j]
        kj = bki[:, j:j + 1]
        idxf = jnp.where(iota_t == tj, kj, idxf)

    onehot = (iota == idxf).astype(jnp.float32)
    sampleT = jax.lax.dot_general(
        musbf, onehot, (((0,), (0,)), ((), ())),
        preferred_element_type=jnp.float32, precision=_HI)
    out_ref[0] = sampleT

    @pl.when(jnp.logical_and(b == 0, m == 0))
    def _init():
        kld_ref[...] = jnp.zeros((1, 1), jnp.float32)
    kld_ref[...] += jnp.reshape(kld_part, (1, 1))


@jax.jit
def kernel(x, mus, scales, log_py_raw):
    b, c, h, w = x.shape
    hw = h * w
    x3 = x.reshape(b, c, hw)
    sc3 = scales.reshape(_M, _K, 1)
    lp3 = log_py_raw.reshape(_M, _K, 1)
    musbf = mus.astype(jnp.bfloat16).astype(jnp.float32)
    scR = jnp.swapaxes(sc3, 1, 2)
    lscR = jnp.log(jnp.clip(scR, _EPS, None))
    lpR = log_py_raw.reshape(_M, 1, _K)

    sample3, kld_acc = pl.pallas_call(
        _body,
        grid=(b, _M),
        in_specs=[
            pl.BlockSpec((1, _D, hw), lambda bi, mi: (bi, mi, 0)),
            pl.BlockSpec((1, _K, _D), lambda bi, mi: (mi, 0, 0)),
            pl.BlockSpec((1, _K, _D), lambda bi, mi: (mi, 0, 0)),
            pl.BlockSpec((1, _K, 1), lambda bi, mi: (mi, 0, 0)),
            pl.BlockSpec((1, _K, 1), lambda bi, mi: (mi, 0, 0)),
            pl.BlockSpec((1, 1, _K), lambda bi, mi: (mi, 0, 0)),
            pl.BlockSpec((1, 1, _K), lambda bi, mi: (mi, 0, 0)),
            pl.BlockSpec((1, 1, _K), lambda bi, mi: (mi, 0, 0)),
        ],
        out_specs=[
            pl.BlockSpec((1, _D, hw), lambda bi, mi: (bi, mi, 0)),
            pl.BlockSpec((1, 1), lambda bi, mi: (0, 0)),
        ],
        out_shape=[
            jax.ShapeDtypeStruct((b, c, hw), jnp.float32),
            jax.ShapeDtypeStruct((1, 1), jnp.float32),
        ],
    )(x3, mus, musbf, sc3, lp3, scR, lscR, lpR)

    sample = sample3.reshape(b, c, h, w)
    kldesum = kld_acc[0, 0] / b
    return (sample, kldesum, jnp.zeros_like(kldesum))


# speedup vs baseline: 1.4440x; 1.4440x over previous
import math

import jax
import jax.numpy as jnp
from jax.experimental import pallas as pl

_M, _K, _D = 8, 256, 32
_HW = 256
_EPS = 1e-05
_C = 0.5 * math.log(2.0 * math.pi)
_DELTA = 1.2e-4
_RCAP = 16
_NEG = -1e30
_HI = jax.lax.Precision.HIGHEST


def _tree8(w):
    parts = []
    for c in range(4):
        t = w[8 * c:8 * c + 8, :]
        t = t[0:4, :] + t[4:8, :]
        t = t[0:2, :] + t[2:4, :]
        t = t[0:1, :] + t[1:2, :]
        parts.append(t)
    return ((parts[0] + parts[1]) + parts[2]) + parts[3]


def _body(x_ref, mus_ref, musbf_ref, sc_ref, lp_ref, scr_ref, lscr_ref,
          lpr_ref, out_ref, kld_ref):
    b = pl.program_id(0)
    m = pl.program_id(1)

    xs = x_ref[0]
    mus = mus_ref[0]
    musbf = musbf_ref[0]
    sc = jnp.clip(sc_ref[0], _EPS, None)
    lp = lp_ref[0]

    a = 0.5 / (sc * sc)
    musq = jnp.sum(mus * mus, axis=1, keepdims=True)
    cst = -_D * (jnp.log(sc) + _C) + lp

    dot = jnp.dot(mus, xs, preferred_element_type=jnp.float32, precision=_HI)
    logits = a * (2.0 * dot - musq) + cst

    colmax = jnp.max(logits, axis=0, keepdims=True)
    shifted = logits - colmax
    e = jnp.exp(shifted)
    se = jnp.sum(e, axis=0, keepdims=True)
    lse_sh = jnp.log(se)
    p = e / se
    lpmax = jnp.max(lp)
    log_prior = lp - (lpmax + jnp.log(jnp.sum(jnp.exp(lp - lpmax))))
    kld_part = jnp.sum(p * (shifted - lse_sh - log_prior))

    iota = jax.lax.broadcasted_iota(jnp.int32, (_K, _HW), 0)
    lcur = logits
    idxs = []
    vals = []
    for _ in range(4):
        vmax = jnp.max(lcur, axis=0, keepdims=True)
        idx = jnp.min(jnp.where(lcur == vmax, iota, _K), axis=0, keepdims=True)
        idxs.append(idx)
        vals.append(vmax)
        lcur = jnp.where(iota == idx, _NEG, lcur)
    gap = vals[0] - vals[1]

    risk = (gap < _DELTA).astype(jnp.float32)
    rank = risk
    for sh in (1, 2, 4, 8, 16, 32, 64, 128):
        rank = rank + jnp.concatenate(
            [jnp.zeros((1, sh), jnp.float32), rank[:, :_HW - sh]], axis=1)
    rank_col = jnp.transpose(rank)
    risk_col = jnp.transpose(risk)
    iota_j = jax.lax.broadcasted_iota(
        jnp.int32, (_HW, _RCAP), 1).astype(jnp.float32)
    selT = jnp.where(rank_col == iota_j + 1.0, risk_col,
                     0.0)
    xsel = jnp.dot(xs, selT, preferred_element_type=jnp.float32,
                   precision=_HI)
    cand4 = jnp.concatenate(idxs, axis=0).astype(jnp.float32)
    codes = jnp.dot(cand4, selT, preferred_element_type=jnp.float32,
                    precision=_HI)
    codesflat = jnp.concatenate(
        [codes[j:j + 1, :] for j in range(4)], axis=1)

    scr = jnp.clip(scr_ref[0], _EPS, None)
    den_row = 2.0 * scr ** 2
    lsc_row = lscr_ref[0]
    lp_row = lpr_ref[0]

    iota_k64 = jax.lax.broadcasted_iota(
        jnp.int32, (_K, 4 * _RCAP), 0).astype(jnp.float32)
    selK = (iota_k64 == codesflat).astype(jnp.float32)
    mus_sel = jax.lax.dot_general(
        mus, selK, (((0,), (0,)), ((), ())),
        preferred_element_type=jnp.float32, precision=_HI)
    den_sel = jnp.dot(den_row, selK, preferred_element_type=jnp.float32,
                      precision=_HI)
    lsc_sel = jnp.dot(lsc_row, selK, preferred_element_type=jnp.float32,
                      precision=_HI)
    lp_sel = jnp.dot(lp_row, selK, preferred_element_type=jnp.float32,
                     precision=_HI)
    xsel4 = jnp.concatenate([xsel] * 4, axis=1)
    diff = xsel4 - mus_sel
    w = -(diff * diff) / den_sel - lsc_sel - _C
    v64 = _tree8(w) + lp_sel

    bv = v64[:, 0:_RCAP]
    bk = codesflat[:, 0:_RCAP]
    for cj in range(1, 4):
        vj = v64[:, cj * _RCAP:(cj + 1) * _RCAP]
        kj = codesflat[:, cj * _RCAP:(cj + 1) * _RCAP]
        better = jnp.logical_or(vj > bv,
                                jnp.logical_and(vj == bv, kj < bk))
        bv = jnp.where(better, vj, bv)
        bk = jnp.where(better, kj, bk)

    patch = jax.lax.dot_general(
        bk, selT, (((1,), (1,)), ((), ())),
        preferred_element_type=jnp.float32, precision=_HI)
    hit = jnp.logical_and(risk > 0, rank <= _RCAP)
    idxf = jnp.where(hit, patch.astype(jnp.int32), idxs[0])

    onehot = (iota == idxf).astype(jnp.float32)
    sampleT = jax.lax.dot_general(
        musbf, onehot, (((0,), (0,)), ((), ())),
        preferred_element_type=jnp.float32, precision=_HI)
    out_ref[0] = sampleT

    @pl.when(jnp.logical_and(b == 0, m == 0))
    def _init():
        kld_ref[...] = jnp.zeros((1, 1), jnp.float32)
    kld_ref[...] += jnp.reshape(kld_part, (1, 1))


@jax.jit
def kernel(x, mus, scales, log_py_raw):
    b, c, h, w = x.shape
    hw = h * w
    x3 = x.reshape(b, c, hw)
    sc3 = scales.reshape(_M, _K, 1)
    lp3 = log_py_raw.reshape(_M, _K, 1)
    musbf = mus.astype(jnp.bfloat16).astype(jnp.float32)
    scR = jnp.swapaxes(sc3, 1, 2)
    lscR = jnp.log(jnp.clip(scR, _EPS, None))
    lpR = log_py_raw.reshape(_M, 1, _K)

    sample3, kld_acc = pl.pallas_call(
        _body,
        grid=(b, _M),
        in_specs=[
            pl.BlockSpec((1, _D, hw), lambda bi, mi: (bi, mi, 0)),
            pl.BlockSpec((1, _K, _D), lambda bi, mi: (mi, 0, 0)),
            pl.BlockSpec((1, _K, _D), lambda bi, mi: (mi, 0, 0)),
            pl.BlockSpec((1, _K, 1), lambda bi, mi: (mi, 0, 0)),
            pl.BlockSpec((1, _K, 1), lambda bi, mi: (mi, 0, 0)),
            pl.BlockSpec((1, 1, _K), lambda bi, mi: (mi, 0, 0)),
            pl.BlockSpec((1, 1, _K), lambda bi, mi: (mi, 0, 0)),
            pl.BlockSpec((1, 1, _K), lambda bi, mi: (mi, 0, 0)),
        ],
        out_specs=[
            pl.BlockSpec((1, _D, hw), lambda bi, mi: (bi, mi, 0)),
            pl.BlockSpec((1, 1), lambda bi, mi: (0, 0)),
        ],
        out_shape=[
            jax.ShapeDtypeStruct((b, c, hw), jnp.float32),
            jax.ShapeDtypeStruct((1, 1), jnp.float32),
        ],
    )(x3, mus, musbf, sc3, lp3, scR, lscR, lpR)

    sample = sample3.reshape(b, c, h, w)
    kldesum = kld_acc[0, 0] / b
    return (sample, kldesum, jnp.zeros_like(kldesum))


# speedup vs baseline: 1.4462x; 1.0015x over previous
import math

import jax
import jax.numpy as jnp
from jax.experimental import pallas as pl

_M, _K, _D = 8, 256, 32
_HW = 256
_EPS = 1e-05
_C = 0.5 * math.log(2.0 * math.pi)
_DELTA = 1.2e-4
_RCAP = 16
_NEG = -1e30
_HI = jax.lax.Precision.HIGHEST


def _tree8(w):
    parts = []
    for c in range(4):
        t = w[8 * c:8 * c + 8, :]
        t = t[0:4, :] + t[4:8, :]
        t = t[0:2, :] + t[2:4, :]
        t = t[0:1, :] + t[1:2, :]
        parts.append(t)
    return ((parts[0] + parts[1]) + parts[2]) + parts[3]


def _body(x_ref, mus_ref, musbf_ref, sc_ref, lp_ref, scr_ref, lscr_ref,
          lpr_ref, out_ref, kld_ref):
    b = pl.program_id(0)
    m = pl.program_id(1)

    xs = x_ref[0]
    mus = mus_ref[0]
    musbf = musbf_ref[0]
    sc = jnp.clip(sc_ref[0], _EPS, None)
    lp = lp_ref[0]

    a = 0.5 / (sc * sc)
    musq = jnp.sum(mus * mus, axis=1, keepdims=True)
    cst = -_D * (jnp.log(sc) + _C) + lp

    dot = jnp.dot(mus, xs, preferred_element_type=jnp.float32, precision=_HI)
    logits = a * (2.0 * dot - musq) + cst

    colmax = jnp.max(logits, axis=0, keepdims=True)
    shifted = logits - colmax
    e = jnp.exp(shifted)
    se = jnp.sum(e, axis=0, keepdims=True)
    lse_sh = jnp.log(se)
    lpmax = jnp.max(lp)
    log_prior = lp - (lpmax + jnp.log(jnp.sum(jnp.exp(lp - lpmax))))
    s1 = jnp.sum(e * (shifted - log_prior), axis=0, keepdims=True)
    kld_part = jnp.sum(s1 / se - lse_sh)

    iota = jax.lax.broadcasted_iota(jnp.int32, (_K, _HW), 0)
    lcur = logits
    idxs = []
    vals = []
    for _ in range(4):
        vmax = jnp.max(lcur, axis=0, keepdims=True)
        idx = jnp.min(jnp.where(lcur == vmax, iota, _K), axis=0, keepdims=True)
        idxs.append(idx)
        vals.append(vmax)
        lcur = jnp.where(iota == idx, _NEG, lcur)
    gap = vals[0] - vals[1]

    risk = (gap < _DELTA).astype(jnp.float32)
    rank = risk
    for sh in (1, 2, 4, 8, 16, 32, 64, 128):
        rank = rank + jnp.concatenate(
            [jnp.zeros((1, sh), jnp.float32), rank[:, :_HW - sh]], axis=1)
    rank_col = jnp.transpose(rank)
    risk_col = jnp.transpose(risk)
    iota_j = jax.lax.broadcasted_iota(
        jnp.int32, (_HW, _RCAP), 1).astype(jnp.float32)
    selT = jnp.where(rank_col == iota_j + 1.0, risk_col,
                     0.0)
    xsel = jnp.dot(xs, selT, preferred_element_type=jnp.float32,
                   precision=_HI)
    cand4 = jnp.concatenate(idxs, axis=0).astype(jnp.float32)
    codes = jnp.dot(cand4, selT, preferred_element_type=jnp.float32,
                    precision=_HI)
    codesflat = jnp.concatenate(
        [codes[j:j + 1, :] for j in range(4)], axis=1)

    scr = jnp.clip(scr_ref[0], _EPS, None)
    den_row = 2.0 * scr ** 2
    lsc_row = lscr_ref[0]
    lp_row = lpr_ref[0]

    iota_k64 = jax.lax.broadcasted_iota(
        jnp.int32, (_K, 4 * _RCAP), 0).astype(jnp.float32)
    selK = (iota_k64 == codesflat).astype(jnp.float32)
    mus_sel = jax.lax.dot_general(
        mus, selK, (((0,), (0,)), ((), ())),
        preferred_element_type=jnp.float32, precision=_HI)
    den_sel = jnp.dot(den_row, selK, preferred_element_type=jnp.float32,
                      precision=_HI)
    lsc_sel = jnp.dot(lsc_row, selK, preferred_element_type=jnp.float32,
                      precision=_HI)
    lp_sel = jnp.dot(lp_row, selK, preferred_element_type=jnp.float32,
                     precision=_HI)
    xsel4 = jnp.concatenate([xsel] * 4, axis=1)
    diff = xsel4 - mus_sel
    w = -(diff * diff) / den_sel - lsc_sel - _C
    v64 = _tree8(w) + lp_sel

    bv = v64[:, 0:_RCAP]
    bk = codesflat[:, 0:_RCAP]
    for cj in range(1, 4):
        vj = v64[:, cj * _RCAP:(cj + 1) * _RCAP]
        kj = codesflat[:, cj * _RCAP:(cj + 1) * _RCAP]
        better = jnp.logical_or(vj > bv,
                                jnp.logical_and(vj == bv, kj < bk))
        bv = jnp.where(better, vj, bv)
        bk = jnp.where(better, kj, bk)

    patch = jax.lax.dot_general(
        bk, selT, (((1,), (1,)), ((), ())),
        preferred_element_type=jnp.float32, precision=_HI)
    hit = jnp.logical_and(risk > 0, rank <= _RCAP)
    idxf = jnp.where(hit, patch.astype(jnp.int32), idxs[0])

    onehot = (iota == idxf).astype(jnp.float32)
    sampleT = jax.lax.dot_general(
        musbf, onehot, (((0,), (0,)), ((), ())),
        preferred_element_type=jnp.float32, precision=_HI)
    out_ref[0] = sampleT

    @pl.when(jnp.logical_and(b == 0, m == 0))
    def _init():
        kld_ref[...] = jnp.zeros((1, 1), jnp.float32)
    kld_ref[...] += jnp.reshape(kld_part, (1, 1))


@jax.jit
def kernel(x, mus, scales, log_py_raw):
    b, c, h, w = x.shape
    hw = h * w
    x3 = x.reshape(b, c, hw)
    sc3 = scales.reshape(_M, _K, 1)
    lp3 = log_py_raw.reshape(_M, _K, 1)
    musbf = mus.astype(jnp.bfloat16).astype(jnp.float32)
    scR = jnp.swapaxes(sc3, 1, 2)
    lscR = jnp.log(jnp.clip(scR, _EPS, None))
    lpR = log_py_raw.reshape(_M, 1, _K)

    sample3, kld_acc = pl.pallas_call(
        _body,
        grid=(b, _M),
        in_specs=[
            pl.BlockSpec((1, _D, hw), lambda bi, mi: (bi, mi, 0)),
            pl.BlockSpec((1, _K, _D), lambda bi, mi: (mi, 0, 0)),
            pl.BlockSpec((1, _K, _D), lambda bi, mi: (mi, 0, 0)),
            pl.BlockSpec((1, _K, 1), lambda bi, mi: (mi, 0, 0)),
            pl.BlockSpec((1, _K, 1), lambda bi, mi: (mi, 0, 0)),
            pl.BlockSpec((1, 1, _K), lambda bi, mi: (mi, 0, 0)),
            pl.BlockSpec((1, 1, _K), lambda bi, mi: (mi, 0, 0)),
            pl.BlockSpec((1, 1, _K), lambda bi, mi: (mi, 0, 0)),
        ],
        out_specs=[
            pl.BlockSpec((1, _D, hw), lambda bi, mi: (bi, mi, 0)),
            pl.BlockSpec((1, 1), lambda bi, mi: (0, 0)),
        ],
        out_shape=[
            jax.ShapeDtypeStruct((b, c, hw), jnp.float32),
            jax.ShapeDtypeStruct((1, 1), jnp.float32),
        ],
    )(x3, mus, musbf, sc3, lp3, scR, lscR, lpR)

    sample = sample3.reshape(b, c, h, w)
    kldesum = kld_acc[0, 0] / b
    return (sample, kldesum, jnp.zeros_like(kldesum))


# speedup vs baseline: 1.4464x; 1.0002x over previous
import math

import jax
import jax.numpy as jnp
from jax.experimental import pallas as pl

_M, _K, _D = 8, 256, 32
_HW = 256
_EPS = 1e-05
_C = 0.5 * math.log(2.0 * math.pi)
_DELTA = 1.2e-4
_RCAP = 16
_NEG = -1e30
_HI = jax.lax.Precision.HIGHEST


def _tree8(w):
    parts = []
    for c in range(4):
        t = w[8 * c:8 * c + 8, :]
        t = t[0:4, :] + t[4:8, :]
        t = t[0:2, :] + t[2:4, :]
        t = t[0:1, :] + t[1:2, :]
        parts.append(t)
    return ((parts[0] + parts[1]) + parts[2]) + parts[3]


def _body(x_ref, mus_ref, musbf_ref, sc_ref, lp_ref, scr_ref, lscr_ref,
          lpr_ref, out_ref, kld_ref):
    b = pl.program_id(0)
    m = pl.program_id(1)

    xs = x_ref[0]
    mus = mus_ref[0]
    musbf = musbf_ref[0]
    sc = jnp.clip(sc_ref[0], _EPS, None)
    lp = lp_ref[0]

    a = 0.5 / (sc * sc)
    musq = jnp.sum(mus * mus, axis=1, keepdims=True)
    cst = -_D * (jnp.log(sc) + _C) + lp

    dot = jnp.dot(mus, xs, preferred_element_type=jnp.float32, precision=_HI)
    logits = a * (2.0 * dot - musq) + cst

    colmax = jnp.max(logits, axis=0, keepdims=True)
    shifted = logits - colmax
    e = jnp.exp(shifted)
    se = jnp.sum(e, axis=0, keepdims=True)
    lse_sh = jnp.log(se)
    p = e / se
    lpmax = jnp.max(lp)
    log_prior = lp - (lpmax + jnp.log(jnp.sum(jnp.exp(lp - lpmax))))
    kld_part = jnp.sum(p * (shifted - lse_sh - log_prior))

    iota = jax.lax.broadcasted_iota(jnp.int32, (_K, _HW), 0)
    lcur = logits
    idxs = []
    vals = []
    for _ in range(4):
        vmax = jnp.max(lcur, axis=0, keepdims=True)
        idx = jnp.min(jnp.where(lcur == vmax, iota, _K), axis=0, keepdims=True)
        idxs.append(idx)
        vals.append(vmax)
        lcur = jnp.where(iota == idx, _NEG, lcur)
    gap = vals[0] - vals[1]

    risk = (gap < _DELTA).astype(jnp.float32)
    rank = risk
    for sh in (1, 2, 4, 8, 16, 32, 64, 128):
        rank = rank + jnp.concatenate(
            [jnp.zeros((1, sh), jnp.float32), rank[:, :_HW - sh]], axis=1)
    rank_col = jnp.transpose(rank)
    risk_col = jnp.transpose(risk)
    iota_j = jax.lax.broadcasted_iota(
        jnp.int32, (_HW, _RCAP), 1).astype(jnp.float32)
    selT = jnp.where(rank_col == iota_j + 1.0, risk_col,
                     0.0)
    xsel = jnp.dot(xs, selT, preferred_element_type=jnp.float32,
                   precision=_HI)
    cand4 = jnp.concatenate(idxs, axis=0).astype(jnp.float32)
    codes = jnp.dot(cand4, selT, preferred_element_type=jnp.float32,
                    precision=_HI)
    codesflat = jnp.concatenate(
        [codes[j:j + 1, :] for j in range(4)], axis=1)

    scr = jnp.clip(scr_ref[0], _EPS, None)
    den_row = 2.0 * scr ** 2
    lsc_row = lscr_ref[0]
    lp_row = lpr_ref[0]

    iota_k64 = jax.lax.broadcasted_iota(
        jnp.int32, (_K, 4 * _RCAP), 0).astype(jnp.float32)
    selK = (iota_k64 == codesflat).astype(jnp.float32)
    mus_sel = jax.lax.dot_general(
        mus, selK, (((0,), (0,)), ((), ())),
        preferred_element_type=jnp.float32, precision=_HI)
    den_sel = jnp.dot(den_row, selK, preferred_element_type=jnp.float32,
                      precision=_HI)
    lsc_sel = jnp.dot(lsc_row, selK, preferred_element_type=jnp.float32,
                      precision=_HI)
    lp_sel = jnp.dot(lp_row, selK, preferred_element_type=jnp.float32,
                     precision=_HI)
    xsel4 = jnp.concatenate([xsel] * 4, axis=1)
    diff = xsel4 - mus_sel
    w = -(diff * diff) / den_sel - lsc_sel - _C
    v64 = _tree8(w) + lp_sel

    bv = v64[:, 0:_RCAP]
    bk = codesflat[:, 0:_RCAP]
    for cj in range(1, 4):
        vj = v64[:, cj * _RCAP:(cj + 1) * _RCAP]
        kj = codesflat[:, cj * _RCAP:(cj + 1) * _RCAP]
        better = jnp.logical_or(vj > bv,
                                jnp.logical_and(vj == bv, kj < bk))
        bv = jnp.where(better, vj, bv)
        bk = jnp.where(better, kj, bk)

    patch = jax.lax.dot_general(
        bk, selT, (((1,), (1,)), ((), ())),
        preferred_element_type=jnp.float32, precision=_HI)
    hit = jnp.logical_and(risk > 0, rank <= _RCAP)
    idxf = jnp.where(hit, patch.astype(jnp.int32), idxs[0])

    onehot = (iota == idxf).astype(jnp.float32)
    sampleT = jax.lax.dot_general(
        musbf, onehot, (((0,), (0,)), ((), ())),
        preferred_element_type=jnp.float32, precision=_HI)
    out_ref[0] = sampleT

    @pl.when(jnp.logical_and(b == 0, m == 0))
    def _init():
        kld_ref[...] = jnp.zeros((1, 1), jnp.float32)
    kld_ref[...] += jnp.reshape(kld_part, (1, 1))


@jax.jit
def kernel(x, mus, scales, log_py_raw):
    b, c, h, w = x.shape
    hw = h * w
    x3 = x.reshape(b, c, hw)
    sc3 = scales.reshape(_M, _K, 1)
    lp3 = log_py_raw.reshape(_M, _K, 1)
    musbf = mus.astype(jnp.bfloat16).astype(jnp.float32)
    scR = jnp.swapaxes(sc3, 1, 2)
    lscR = jnp.log(jnp.clip(scR, _EPS, None))
    lpR = log_py_raw.reshape(_M, 1, _K)

    sample3, kld_acc = pl.pallas_call(
        _body,
        grid=(b, _M),
        in_specs=[
            pl.BlockSpec((1, _D, hw), lambda bi, mi: (bi, mi, 0)),
            pl.BlockSpec((1, _K, _D), lambda bi, mi: (mi, 0, 0)),
            pl.BlockSpec((1, _K, _D), lambda bi, mi: (mi, 0, 0)),
            pl.BlockSpec((1, _K, 1), lambda bi, mi: (mi, 0, 0)),
            pl.BlockSpec((1, _K, 1), lambda bi, mi: (mi, 0, 0)),
            pl.BlockSpec((1, 1, _K), lambda bi, mi: (mi, 0, 0)),
            pl.BlockSpec((1, 1, _K), lambda bi, mi: (mi, 0, 0)),
            pl.BlockSpec((1, 1, _K), lambda bi, mi: (mi, 0, 0)),
        ],
        out_specs=[
            pl.BlockSpec((1, _D, hw), lambda bi, mi: (bi, mi, 0)),
            pl.BlockSpec((1, 1), lambda bi, mi: (0, 0)),
        ],
        out_shape=[
            jax.ShapeDtypeStruct((b, c, hw), jnp.float32),
            jax.ShapeDtypeStruct((1, 1), jnp.float32),
        ],
    )(x3, mus, musbf, sc3, lp3, scR, lscR, lpR)

    sample = sample3.reshape(b, c, h, w)
    kldesum = kld_acc[0, 0] / b
    return (sample, kldesum, jnp.zeros_like(kldesum))
